# baseline (speedup 1.0000x reference)
import functools

import numpy as np
import jax
import jax.numpy as jnp
from jax import lax
from jax.experimental import pallas as pl
from jax.experimental.pallas import tpu as pltpu

NUM_CLASSES = 133
NUM_TOTAL = NUM_CLASSES + 1
Q = 200
TOPK = 100
IN_HW = 128
OUT_HW = 512


def _topk_kernel(x_ref, vals_ref, lab_ref, qid_ref):
    x = x_ref[...]
    m = jnp.max(x, axis=1, keepdims=True)
    e = jnp.exp(x - m)
    s = e / jnp.sum(e, axis=1, keepdims=True)

    col = lax.broadcasted_iota(jnp.int32, (Q, NUM_TOTAL), 1)
    row = lax.broadcasted_iota(jnp.int32, (Q, NUM_TOTAL), 0)
    fi = row * NUM_CLASSES + col
    valid = col < NUM_CLASSES
    sv0 = jnp.where(valid, s, -1.0)

    def body(k, sv):
        mv = jnp.max(jnp.max(sv, axis=1, keepdims=True), axis=0, keepdims=True)
        cand = jnp.where(sv == mv, fi, jnp.int32(2**30))
        ii = jnp.min(jnp.min(cand, axis=1, keepdims=True), axis=0, keepdims=True)
        vals_ref[pl.ds(k, 1), :] = jnp.broadcast_to(mv, (1, 128))
        lab_ref[pl.ds(k, 1), :] = jnp.broadcast_to(ii % NUM_CLASSES, (1, 128))
        qid_ref[pl.ds(k, 1), :] = jnp.broadcast_to(ii // NUM_CLASSES, (1, 128))
        return jnp.where(fi == ii, -1.0, sv)

    lax.fori_loop(0, TOPK, body, sv0)


def _run_topk(mask_cls2d):
    vals, lab, qid = pl.pallas_call(
        _topk_kernel,
        out_shape=(
            jax.ShapeDtypeStruct((TOPK, 128), jnp.float32),
            jax.ShapeDtypeStruct((TOPK, 128), jnp.int32),
            jax.ShapeDtypeStruct((TOPK, 128), jnp.int32),
        ),
    )(mask_cls2d)
    return vals[:, 0], lab[:, 0], qid[:, 0]


def _mask_kernel(qid_ref, x_ref, u_ref, ut_ref, tv_ref, pred_ref, score_ref):
    x = x_ref[0]
    u = u_ref[...]
    ut = ut_ref[...]
    y = jnp.dot(u, x, preferred_element_type=jnp.float32,
                precision=lax.Precision.HIGHEST)
    z = jnp.dot(y, ut, preferred_element_type=jnp.float32,
                precision=lax.Precision.HIGHEST)
    sig = jax.nn.sigmoid(z)
    pos = z > 0.0
    pred_ref[0] = pos
    posf = pos.astype(jnp.float32)
    num = jnp.sum(sig * posf)
    den = jnp.sum(posf)
    score_ref[0, 0, :] = tv_ref[0, 0, :] * num / (den + 1e-6)


def _bilinear_matrix(out_size, in_size):
    sample = (np.arange(out_size) + 0.5) * (in_size / out_size) - 0.5
    w = np.maximum(0.0, 1.0 - np.abs(sample[:, None] - np.arange(in_size)[None, :]))
    w = w / w.sum(axis=1, keepdims=True)
    return w.astype(np.float32)


def _run_masks(mask_pred3d, qid, top_vals):
    u = jnp.asarray(_bilinear_matrix(OUT_HW, IN_HW))
    ut = u.T
    tv = jnp.broadcast_to(top_vals[:, None, None], (TOPK, 1, 128))
    grid_spec = pltpu.PrefetchScalarGridSpec(
        num_scalar_prefetch=1,
        grid=(TOPK,),
        in_specs=[
            pl.BlockSpec((1, IN_HW, IN_HW), lambda i, qid_ref: (qid_ref[i], 0, 0)),
            pl.BlockSpec((OUT_HW, IN_HW), lambda i, qid_ref: (0, 0)),
            pl.BlockSpec((IN_HW, OUT_HW), lambda i, qid_ref: (0, 0)),
            pl.BlockSpec((1, 1, 128), lambda i, qid_ref: (i, 0, 0)),
        ],
        out_specs=[
            pl.BlockSpec((1, OUT_HW, OUT_HW), lambda i, qid_ref: (i, 0, 0)),
            pl.BlockSpec((1, 1, 128), lambda i, qid_ref: (i, 0, 0)),
        ],
    )
    pred, score = pl.pallas_call(
        _mask_kernel,
        grid_spec=grid_spec,
        out_shape=(
            jax.ShapeDtypeStruct((TOPK, OUT_HW, OUT_HW), jnp.bool_),
            jax.ShapeDtypeStruct((TOPK, 1, 128), jnp.float32),
        ),
    )(qid, mask_pred3d, u, ut, tv)
    return pred, score[:, 0, 0]


def kernel(mask_cls, mask_pred):
    mask_cls2d = mask_cls.reshape(Q, NUM_TOTAL)
    top_vals, labels, qid = _run_topk(mask_cls2d)
    pred_masks, final_scores = _run_masks(mask_pred[0], qid, top_vals)
    return final_scores, labels, pred_masks


# speedup vs baseline: 1.0548x; 1.0548x over previous
import functools

import numpy as np
import jax
import jax.numpy as jnp
from jax import lax
from jax.experimental import pallas as pl
from jax.experimental.pallas import tpu as pltpu

NUM_CLASSES = 133
NUM_TOTAL = NUM_CLASSES + 1
Q = 200
TOPK = 100
IN_HW = 128
OUT_HW = 512


_BIG = 2**30


def _topk_kernel(x_ref, vals3_ref, lab_ref, qid_ref, sv_ref):
    x = x_ref[...]
    m = jnp.max(x, axis=1, keepdims=True)
    e = jnp.exp(x - m)
    s = e / jnp.sum(e, axis=1, keepdims=True)

    col = lax.broadcasted_iota(jnp.int32, (Q, NUM_TOTAL), 1)
    valid = col < NUM_CLASSES
    sv = jnp.where(valid, s, -1.0)
    sv_ref[...] = sv

    rm0 = jnp.max(sv, axis=1)
    ra0 = jnp.min(jnp.where(sv == rm0[:, None], col, _BIG), axis=1)
    pad = 256 - Q
    rm = jnp.concatenate([rm0, jnp.full((pad,), -2.0, jnp.float32)]).reshape(2, 128)
    ra = jnp.concatenate([ra0, jnp.full((pad,), _BIG, jnp.int32)]).reshape(2, 128)
    rid = lax.broadcasted_iota(jnp.int32, (2, 128), 0) * 128 + \
        lax.broadcasted_iota(jnp.int32, (2, 128), 1)
    colid = lax.broadcasted_iota(jnp.int32, (1, NUM_TOTAL), 1)

    def body(k, carry):
        rm, ra = carry
        mv = jnp.max(jnp.max(rm, axis=1, keepdims=True), axis=0, keepdims=True)
        hit = rm == mv
        r = jnp.min(jnp.min(jnp.where(hit, rid, _BIG), axis=1, keepdims=True),
                    axis=0, keepdims=True)
        sel = rid == r
        c = jnp.min(jnp.min(jnp.where(sel, ra, _BIG), axis=1, keepdims=True),
                    axis=0, keepdims=True)
        vals3_ref[pl.ds(k, 1), :, :] = jnp.broadcast_to(mv, (1, 1, 128))
        lab_ref[k] = c[0, 0]
        qid_ref[k] = r[0, 0]
        r_s = r[0, 0]
        row = sv_ref[pl.ds(r_s, 1), :]
        row = jnp.where(colid == c, -1.0, row)
        sv_ref[pl.ds(r_s, 1), :] = row
        nm = jnp.max(row, axis=1, keepdims=True)
        na = jnp.min(jnp.where(row == nm, colid, _BIG), axis=1, keepdims=True)
        return jnp.where(sel, nm, rm), jnp.where(sel, na, ra)

    lax.fori_loop(0, TOPK, body, (rm, ra), unroll=2)


def _run_topk(mask_cls2d):
    vals3, lab, qid = pl.pallas_call(
        _topk_kernel,
        out_shape=(
            jax.ShapeDtypeStruct((TOPK, 1, 128), jnp.float32),
            jax.ShapeDtypeStruct((TOPK,), jnp.int32),
            jax.ShapeDtypeStruct((TOPK,), jnp.int32),
        ),
        out_specs=(
            pl.BlockSpec((TOPK, 1, 128), lambda: (0, 0, 0)),
            pl.BlockSpec(memory_space=pltpu.SMEM),
            pl.BlockSpec(memory_space=pltpu.SMEM),
        ),
        scratch_shapes=[pltpu.VMEM((Q, NUM_TOTAL), jnp.float32)],
    )(mask_cls2d)
    return vals3, lab, qid


def _split3_bf16(a):
    ah = a.astype(jnp.bfloat16)
    r1 = a - ah.astype(jnp.float32)
    am = r1.astype(jnp.bfloat16)
    al = (r1 - am.astype(jnp.float32)).astype(jnp.bfloat16)
    return ah, am, al


def _mask_kernel(qid_ref, xa_ref, xb_ref, u_ref, ut_ref, tv_ref,
                 pred_ref, score_ref):
    u = u_ref[...]
    ut = ut_ref[...]
    ones_c = jnp.ones((1, OUT_HW), jnp.bfloat16)
    for j, xr in enumerate((xa_ref, xb_ref)):
        x = xr[0]
        xh, xm, xl = _split3_bf16(x)
        y = (jnp.dot(u, xh, preferred_element_type=jnp.float32)
             + jnp.dot(u, xm, preferred_element_type=jnp.float32)
             + jnp.dot(u, xl, preferred_element_type=jnp.float32))
        yh, ym, yl = _split3_bf16(y)
        z = (jnp.dot(yh, ut, preferred_element_type=jnp.float32)
             + jnp.dot(ym, ut, preferred_element_type=jnp.float32)
             + jnp.dot(yl, ut, preferred_element_type=jnp.float32))
        sig = jax.nn.sigmoid(z)
        pos = z > 0.0
        pred_ref[j] = pos
        sigb = jnp.where(pos, sig, 0.0).astype(jnp.bfloat16)
        posb = pos.astype(jnp.bfloat16)
        num = jnp.sum(jnp.dot(ones_c, sigb, preferred_element_type=jnp.float32))
        den = jnp.sum(jnp.dot(ones_c, posb, preferred_element_type=jnp.float32))
        score_ref[j, 0, :] = tv_ref[j, 0, :] * num / (den + 1e-6)


def _bilinear_matrix(out_size, in_size):
    sample = (np.arange(out_size) + 0.5) * (in_size / out_size) - 0.5
    w = np.maximum(0.0, 1.0 - np.abs(sample[:, None] - np.arange(in_size)[None, :]))
    w = w / w.sum(axis=1, keepdims=True)
    return w.astype(np.float32)


def _run_masks(mask_pred3d, qid, tv):
    u_np = _bilinear_matrix(OUT_HW, IN_HW)
    u = jnp.asarray(u_np, dtype=jnp.bfloat16)
    ut = jnp.asarray(u_np.T, dtype=jnp.bfloat16)
    grid_spec = pltpu.PrefetchScalarGridSpec(
        num_scalar_prefetch=1,
        grid=(TOPK // 2,),
        in_specs=[
            pl.BlockSpec((1, IN_HW, IN_HW),
                         lambda i, qid_ref: (qid_ref[2 * i], 0, 0)),
            pl.BlockSpec((1, IN_HW, IN_HW),
                         lambda i, qid_ref: (qid_ref[2 * i + 1], 0, 0)),
            pl.BlockSpec((OUT_HW, IN_HW), lambda i, qid_ref: (0, 0)),
            pl.BlockSpec((IN_HW, OUT_HW), lambda i, qid_ref: (0, 0)),
            pl.BlockSpec((2, 1, 128), lambda i, qid_ref: (i, 0, 0)),
        ],
        out_specs=[
            pl.BlockSpec((2, OUT_HW, OUT_HW), lambda i, qid_ref: (i, 0, 0)),
            pl.BlockSpec((2, 1, 128), lambda i, qid_ref: (i, 0, 0)),
        ],
    )
    pred, score = pl.pallas_call(
        _mask_kernel,
        grid_spec=grid_spec,
        out_shape=(
            jax.ShapeDtypeStruct((TOPK, OUT_HW, OUT_HW), jnp.bool_),
            jax.ShapeDtypeStruct((TOPK, 1, 128), jnp.float32),
        ),
    )(qid, mask_pred3d, mask_pred3d, u, ut, tv)
    return pred, score[:, 0, 0]


def kernel(mask_cls, mask_pred):
    mask_cls2d = mask_cls.reshape(Q, NUM_TOTAL)
    vals3, labels, qid = _run_topk(mask_cls2d)
    pred_masks, final_scores = _run_masks(mask_pred[0], qid, vals3)
    return final_scores, labels, pred_masks


# speedup vs baseline: 1.3218x; 1.2532x over previous
import functools

import numpy as np
import jax
import jax.numpy as jnp
from jax import lax
from jax.experimental import pallas as pl
from jax.experimental.pallas import tpu as pltpu

NUM_CLASSES = 133
NUM_TOTAL = NUM_CLASSES + 1
Q = 200
TOPK = 100
IN_HW = 128
OUT_HW = 512


_BIG = 2**30


def _topk_kernel(x_ref, vals3_ref, lab_ref, qid_ref, sv_ref):
    x = x_ref[...]
    m = jnp.max(x, axis=1, keepdims=True)
    e = jnp.exp(x - m)
    s = e / jnp.sum(e, axis=1, keepdims=True)

    col = lax.broadcasted_iota(jnp.int32, (Q, NUM_TOTAL), 1)
    valid = col < NUM_CLASSES
    sv = jnp.where(valid, s, -1.0)
    sv_ref[...] = sv

    rm0 = jnp.max(sv, axis=1)
    ra0 = jnp.min(jnp.where(sv == rm0[:, None], col, _BIG), axis=1)
    pad = 256 - Q
    rm = jnp.concatenate([rm0, jnp.full((pad,), -2.0, jnp.float32)]).reshape(2, 128)
    ra = jnp.concatenate([ra0, jnp.full((pad,), _BIG, jnp.int32)]).reshape(2, 128)
    rid = lax.broadcasted_iota(jnp.int32, (2, 128), 0) * 128 + \
        lax.broadcasted_iota(jnp.int32, (2, 128), 1)
    colid = lax.broadcasted_iota(jnp.int32, (1, NUM_TOTAL), 1)

    def body(k, carry):
        rm, ra = carry
        mv = jnp.max(jnp.max(rm, axis=1, keepdims=True), axis=0, keepdims=True)
        hit = rm == mv
        r = jnp.min(jnp.min(jnp.where(hit, rid, _BIG), axis=1, keepdims=True),
                    axis=0, keepdims=True)
        sel = rid == r
        c = jnp.min(jnp.min(jnp.where(sel, ra, _BIG), axis=1, keepdims=True),
                    axis=0, keepdims=True)
        vals3_ref[pl.ds(k, 1), :, :] = jnp.broadcast_to(mv, (1, 1, 128))
        lab_ref[k] = c[0, 0]
        qid_ref[k] = r[0, 0]
        r_s = r[0, 0]
        row = sv_ref[pl.ds(r_s, 1), :]
        row = jnp.where(colid == c, -1.0, row)
        sv_ref[pl.ds(r_s, 1), :] = row
        nm = jnp.max(row, axis=1, keepdims=True)
        na = jnp.min(jnp.where(row == nm, colid, _BIG), axis=1, keepdims=True)
        return jnp.where(sel, nm, rm), jnp.where(sel, na, ra)

    lax.fori_loop(0, TOPK, body, (rm, ra), unroll=2)


def _run_topk(mask_cls2d):
    vals3, lab, qid = pl.pallas_call(
        _topk_kernel,
        out_shape=(
            jax.ShapeDtypeStruct((TOPK, 1, 128), jnp.float32),
            jax.ShapeDtypeStruct((TOPK,), jnp.int32),
            jax.ShapeDtypeStruct((TOPK,), jnp.int32),
        ),
        out_specs=(
            pl.BlockSpec((TOPK, 1, 128), lambda: (0, 0, 0)),
            pl.BlockSpec(memory_space=pltpu.SMEM),
            pl.BlockSpec(memory_space=pltpu.SMEM),
        ),
        scratch_shapes=[pltpu.VMEM((Q, NUM_TOTAL), jnp.float32)],
    )(mask_cls2d)
    return vals3, lab, qid


def _split3_bf16(a):
    ah = a.astype(jnp.bfloat16)
    r1 = a - ah.astype(jnp.float32)
    am = r1.astype(jnp.bfloat16)
    al = (r1 - am.astype(jnp.float32)).astype(jnp.bfloat16)
    return ah, am, al


def _mask_kernel(qid_ref, xa_ref, xb_ref, u_ref, ut_ref, tv_ref,
                 pred_ref, score_ref):
    u = u_ref[...]
    ut = ut_ref[...]
    for j, xr in enumerate((xa_ref, xb_ref)):
        x = xr[0]
        xh, xm, xl = _split3_bf16(x)
        y = (jnp.dot(u, xh, preferred_element_type=jnp.float32)
             + jnp.dot(u, xm, preferred_element_type=jnp.float32)
             + jnp.dot(u, xl, preferred_element_type=jnp.float32))
        yh, ym, yl = _split3_bf16(y)
        z = (jnp.dot(yh, ut, preferred_element_type=jnp.float32)
             + jnp.dot(ym, ut, preferred_element_type=jnp.float32)
             + jnp.dot(yl, ut, preferred_element_type=jnp.float32))
        sig = jax.nn.sigmoid(z)
        pos = z > 0.0
        pred_ref[j] = pos
        posf = pos.astype(jnp.float32)
        num = jnp.sum(sig * posf)
        den = jnp.sum(posf)
        score_ref[j, 0, :] = tv_ref[j, 0, :] * num / (den + 1e-6)


def _bilinear_matrix(out_size, in_size):
    sample = (np.arange(out_size) + 0.5) * (in_size / out_size) - 0.5
    w = np.maximum(0.0, 1.0 - np.abs(sample[:, None] - np.arange(in_size)[None, :]))
    w = w / w.sum(axis=1, keepdims=True)
    return w.astype(np.float32)


def _run_masks(mask_pred3d, qid, tv):
    u_np = _bilinear_matrix(OUT_HW, IN_HW)
    u = jnp.asarray(u_np, dtype=jnp.bfloat16)
    ut = jnp.asarray(u_np.T, dtype=jnp.bfloat16)
    grid_spec = pltpu.PrefetchScalarGridSpec(
        num_scalar_prefetch=1,
        grid=(TOPK // 2,),
        in_specs=[
            pl.BlockSpec((1, IN_HW, IN_HW),
                         lambda i, qid_ref: (qid_ref[2 * i], 0, 0)),
            pl.BlockSpec((1, IN_HW, IN_HW),
                         lambda i, qid_ref: (qid_ref[2 * i + 1], 0, 0)),
            pl.BlockSpec((OUT_HW, IN_HW), lambda i, qid_ref: (0, 0)),
            pl.BlockSpec((IN_HW, OUT_HW), lambda i, qid_ref: (0, 0)),
            pl.BlockSpec((2, 1, 128), lambda i, qid_ref: (i, 0, 0)),
        ],
        out_specs=[
            pl.BlockSpec((2, OUT_HW, OUT_HW), lambda i, qid_ref: (i, 0, 0)),
            pl.BlockSpec((2, 1, 128), lambda i, qid_ref: (i, 0, 0)),
        ],
    )
    pred, score = pl.pallas_call(
        _mask_kernel,
        grid_spec=grid_spec,
        out_shape=(
            jax.ShapeDtypeStruct((TOPK, OUT_HW, OUT_HW), jnp.bool_),
            jax.ShapeDtypeStruct((TOPK, 1, 128), jnp.float32),
        ),
    )(qid, mask_pred3d, mask_pred3d, u, ut, tv)
    return pred, score[:, 0, 0]


def kernel(mask_cls, mask_pred):
    mask_cls2d = mask_cls.reshape(Q, NUM_TOTAL)
    vals3, labels, qid = _run_topk(mask_cls2d)
    pred_masks, final_scores = _run_masks(mask_pred[0], qid, vals3)
    return final_scores, labels, pred_masks


# speedup vs baseline: 1.3754x; 1.0405x over previous
import functools

import numpy as np
import jax
import jax.numpy as jnp
from jax import lax
from jax.experimental import pallas as pl
from jax.experimental.pallas import tpu as pltpu

NUM_CLASSES = 133
NUM_TOTAL = NUM_CLASSES + 1
Q = 200
TOPK = 100
IN_HW = 128
OUT_HW = 512


_BIG = 2**30


def _topk_kernel(x_ref, vals3_ref, lab_ref, qid_ref, sv_ref):
    x = x_ref[...]
    m = jnp.max(x, axis=1, keepdims=True)
    e = jnp.exp(x - m)
    s = e / jnp.sum(e, axis=1, keepdims=True)

    col = lax.broadcasted_iota(jnp.int32, (Q, NUM_TOTAL), 1)
    valid = col < NUM_CLASSES
    sv = jnp.where(valid, s, -1.0)
    sv_ref[...] = sv

    rm0 = jnp.max(sv, axis=1)
    ra0 = jnp.min(jnp.where(sv == rm0[:, None], col, _BIG), axis=1)
    pad = 256 - Q
    rm = jnp.concatenate([rm0, jnp.full((pad,), -2.0, jnp.float32)]).reshape(2, 128)
    ra = jnp.concatenate([ra0, jnp.full((pad,), _BIG, jnp.int32)]).reshape(2, 128)
    rid = lax.broadcasted_iota(jnp.int32, (2, 128), 0) * 128 + \
        lax.broadcasted_iota(jnp.int32, (2, 128), 1)
    colid = lax.broadcasted_iota(jnp.int32, (1, NUM_TOTAL), 1)

    def body(k, carry):
        rm, ra = carry
        mv = jnp.max(jnp.max(rm, axis=1, keepdims=True), axis=0, keepdims=True)
        hit = rm == mv
        r = jnp.min(jnp.min(jnp.where(hit, rid, _BIG), axis=1, keepdims=True),
                    axis=0, keepdims=True)
        sel = rid == r
        c = jnp.min(jnp.min(jnp.where(sel, ra, _BIG), axis=1, keepdims=True),
                    axis=0, keepdims=True)
        vals3_ref[pl.ds(k, 1), :, :] = jnp.broadcast_to(mv, (1, 1, 128))
        lab_ref[k] = c[0, 0]
        qid_ref[k] = r[0, 0]
        r_s = r[0, 0]
        row = sv_ref[pl.ds(r_s, 1), :]
        row = jnp.where(colid == c, -1.0, row)
        sv_ref[pl.ds(r_s, 1), :] = row
        nm = jnp.max(row, axis=1, keepdims=True)
        na = jnp.min(jnp.where(row == nm, colid, _BIG), axis=1, keepdims=True)
        return jnp.where(sel, nm, rm), jnp.where(sel, na, ra)

    lax.fori_loop(0, TOPK, body, (rm, ra), unroll=2)


def _run_topk(mask_cls2d):
    vals3, lab, qid = pl.pallas_call(
        _topk_kernel,
        out_shape=(
            jax.ShapeDtypeStruct((TOPK, 1, 128), jnp.float32),
            jax.ShapeDtypeStruct((TOPK,), jnp.int32),
            jax.ShapeDtypeStruct((TOPK,), jnp.int32),
        ),
        out_specs=(
            pl.BlockSpec((TOPK, 1, 128), lambda: (0, 0, 0)),
            pl.BlockSpec(memory_space=pltpu.SMEM),
            pl.BlockSpec(memory_space=pltpu.SMEM),
        ),
        scratch_shapes=[pltpu.VMEM((Q, NUM_TOTAL), jnp.float32)],
    )(mask_cls2d)
    return vals3, lab, qid


def _split3_bf16(a):
    ah = a.astype(jnp.bfloat16)
    r1 = a - ah.astype(jnp.float32)
    am = r1.astype(jnp.bfloat16)
    al = (r1 - am.astype(jnp.float32)).astype(jnp.bfloat16)
    return ah, am, al


def _mask_kernel(qid_ref, xa_ref, xb_ref, xc_ref, xd_ref, u_ref, ut_ref,
                 tv_ref, pred_ref, score_ref):
    u = u_ref[...]
    ut = ut_ref[...]
    for j, xr in enumerate((xa_ref, xb_ref, xc_ref, xd_ref)):
        x = xr[0]
        xh, xm, xl = _split3_bf16(x)
        y = (jnp.dot(u, xh, preferred_element_type=jnp.float32)
             + jnp.dot(u, xm, preferred_element_type=jnp.float32)
             + jnp.dot(u, xl, preferred_element_type=jnp.float32))
        yh, ym, yl = _split3_bf16(y)
        z = (jnp.dot(yh, ut, preferred_element_type=jnp.float32)
             + jnp.dot(ym, ut, preferred_element_type=jnp.float32)
             + jnp.dot(yl, ut, preferred_element_type=jnp.float32))
        sig = jax.nn.sigmoid(z)
        pos = z > 0.0
        pred_ref[j] = pos
        posf = pos.astype(jnp.float32)
        num = jnp.sum(sig * posf)
        den = jnp.sum(posf)
        score_ref[j, 0, :] = tv_ref[j, 0, :] * num / (den + 1e-6)


def _bilinear_matrix(out_size, in_size):
    sample = (np.arange(out_size) + 0.5) * (in_size / out_size) - 0.5
    w = np.maximum(0.0, 1.0 - np.abs(sample[:, None] - np.arange(in_size)[None, :]))
    w = w / w.sum(axis=1, keepdims=True)
    return w.astype(np.float32)


def _run_masks(mask_pred3d, qid, tv):
    u_np = _bilinear_matrix(OUT_HW, IN_HW)
    u = jnp.asarray(u_np, dtype=jnp.bfloat16)
    ut = jnp.asarray(u_np.T, dtype=jnp.bfloat16)
    grid_spec = pltpu.PrefetchScalarGridSpec(
        num_scalar_prefetch=1,
        grid=(TOPK // 4,),
        in_specs=[
            pl.BlockSpec((1, IN_HW, IN_HW),
                         lambda i, qid_ref: (qid_ref[4 * i], 0, 0)),
            pl.BlockSpec((1, IN_HW, IN_HW),
                         lambda i, qid_ref: (qid_ref[4 * i + 1], 0, 0)),
            pl.BlockSpec((1, IN_HW, IN_HW),
                         lambda i, qid_ref: (qid_ref[4 * i + 2], 0, 0)),
            pl.BlockSpec((1, IN_HW, IN_HW),
                         lambda i, qid_ref: (qid_ref[4 * i + 3], 0, 0)),
            pl.BlockSpec((OUT_HW, IN_HW), lambda i, qid_ref: (0, 0)),
            pl.BlockSpec((IN_HW, OUT_HW), lambda i, qid_ref: (0, 0)),
            pl.BlockSpec((4, 1, 128), lambda i, qid_ref: (i, 0, 0)),
        ],
        out_specs=[
            pl.BlockSpec((4, OUT_HW, OUT_HW), lambda i, qid_ref: (i, 0, 0)),
            pl.BlockSpec((4, 1, 128), lambda i, qid_ref: (i, 0, 0)),
        ],
    )
    pred, score = pl.pallas_call(
        _mask_kernel,
        grid_spec=grid_spec,
        out_shape=(
            jax.ShapeDtypeStruct((TOPK, OUT_HW, OUT_HW), jnp.bool_),
            jax.ShapeDtypeStruct((TOPK, 1, 128), jnp.float32),
        ),
    )(qid, mask_pred3d, mask_pred3d, mask_pred3d, mask_pred3d, u, ut, tv)
    return pred, score[:, 0, 0]


def kernel(mask_cls, mask_pred):
    mask_cls2d = mask_cls.reshape(Q, NUM_TOTAL)
    vals3, labels, qid = _run_topk(mask_cls2d)
    pred_masks, final_scores = _run_masks(mask_pred[0], qid, vals3)
    return final_scores, labels, pred_masks


# speedup vs baseline: 1.4001x; 1.0180x over previous
import functools

import numpy as np
import jax
import jax.numpy as jnp
from jax import lax
from jax.experimental import pallas as pl
from jax.experimental.pallas import tpu as pltpu

NUM_CLASSES = 133
NUM_TOTAL = NUM_CLASSES + 1
Q = 200
TOPK = 100
IN_HW = 128
OUT_HW = 512
MASKS_PER_STEP = 10


_BIG = 2**30


def _topk_kernel(x_ref, vals3_ref, lab_ref, qid_ref, sv_ref):
    x = x_ref[...]
    m = jnp.max(x, axis=1, keepdims=True)
    e = jnp.exp(x - m)
    s = e / jnp.sum(e, axis=1, keepdims=True)

    col = lax.broadcasted_iota(jnp.int32, (Q, NUM_TOTAL), 1)
    valid = col < NUM_CLASSES
    sv = jnp.where(valid, s, -1.0)
    sv_ref[...] = sv

    rm0 = jnp.max(sv, axis=1)
    ra0 = jnp.min(jnp.where(sv == rm0[:, None], col, _BIG), axis=1)
    pad = 256 - Q
    rm = jnp.concatenate([rm0, jnp.full((pad,), -2.0, jnp.float32)]).reshape(2, 128)
    ra = jnp.concatenate([ra0, jnp.full((pad,), _BIG, jnp.int32)]).reshape(2, 128)
    rid = lax.broadcasted_iota(jnp.int32, (2, 128), 0) * 128 + \
        lax.broadcasted_iota(jnp.int32, (2, 128), 1)
    colid = lax.broadcasted_iota(jnp.int32, (1, NUM_TOTAL), 1)

    def body(k, carry):
        rm, ra = carry
        mv = jnp.max(jnp.max(rm, axis=1, keepdims=True), axis=0, keepdims=True)
        hit = rm == mv
        r = jnp.min(jnp.min(jnp.where(hit, rid, _BIG), axis=1, keepdims=True),
                    axis=0, keepdims=True)
        sel = rid == r
        c = jnp.min(jnp.min(jnp.where(sel, ra, _BIG), axis=1, keepdims=True),
                    axis=0, keepdims=True)
        vals3_ref[pl.ds(k, 1), :, :] = jnp.broadcast_to(mv, (1, 1, 128))
        lab_ref[k] = c[0, 0]
        qid_ref[k] = r[0, 0]
        r_s = r[0, 0]
        row = sv_ref[pl.ds(r_s, 1), :]
        row = jnp.where(colid == c, -1.0, row)
        sv_ref[pl.ds(r_s, 1), :] = row
        nm = jnp.max(row, axis=1, keepdims=True)
        na = jnp.min(jnp.where(row == nm, colid, _BIG), axis=1, keepdims=True)
        return jnp.where(sel, nm, rm), jnp.where(sel, na, ra)

    lax.fori_loop(0, TOPK, body, (rm, ra), unroll=2)


def _run_topk(mask_cls2d):
    vals3, lab, qid = pl.pallas_call(
        _topk_kernel,
        out_shape=(
            jax.ShapeDtypeStruct((TOPK, 1, 128), jnp.float32),
            jax.ShapeDtypeStruct((TOPK,), jnp.int32),
            jax.ShapeDtypeStruct((TOPK,), jnp.int32),
        ),
        out_specs=(
            pl.BlockSpec((TOPK, 1, 128), lambda: (0, 0, 0)),
            pl.BlockSpec(memory_space=pltpu.SMEM),
            pl.BlockSpec(memory_space=pltpu.SMEM),
        ),
        scratch_shapes=[pltpu.VMEM((Q, NUM_TOTAL), jnp.float32)],
    )(mask_cls2d)
    return vals3, lab, qid


def _split3_bf16(a):
    ah = a.astype(jnp.bfloat16)
    r1 = a - ah.astype(jnp.float32)
    am = r1.astype(jnp.bfloat16)
    al = (r1 - am.astype(jnp.float32)).astype(jnp.bfloat16)
    return ah, am, al


def _mask_kernel(qid_ref, *refs):
    xrefs = refs[:MASKS_PER_STEP]
    u_ref, ut_ref, tv_ref, pred_ref, score_ref = refs[MASKS_PER_STEP:]
    u = u_ref[...]
    ut = ut_ref[...]
    for j, xr in enumerate(xrefs):
        x = xr[0]
        xh, xm, xl = _split3_bf16(x)
        y = (jnp.dot(u, xh, preferred_element_type=jnp.float32)
             + jnp.dot(u, xm, preferred_element_type=jnp.float32)
             + jnp.dot(u, xl, preferred_element_type=jnp.float32))
        yh, ym, yl = _split3_bf16(y)
        z = (jnp.dot(yh, ut, preferred_element_type=jnp.float32)
             + jnp.dot(ym, ut, preferred_element_type=jnp.float32)
             + jnp.dot(yl, ut, preferred_element_type=jnp.float32))
        sig = jax.nn.sigmoid(z)
        pos = z > 0.0
        pred_ref[j] = pos
        posf = pos.astype(jnp.float32)
        num = jnp.sum(sig * posf)
        den = jnp.sum(posf)
        score_ref[j, 0, :] = tv_ref[j, 0, :] * num / (den + 1e-6)


def _bilinear_matrix(out_size, in_size):
    sample = (np.arange(out_size) + 0.5) * (in_size / out_size) - 0.5
    w = np.maximum(0.0, 1.0 - np.abs(sample[:, None] - np.arange(in_size)[None, :]))
    w = w / w.sum(axis=1, keepdims=True)
    return w.astype(np.float32)


def _run_masks(mask_pred3d, qid, tv):
    u_np = _bilinear_matrix(OUT_HW, IN_HW)
    u = jnp.asarray(u_np, dtype=jnp.bfloat16)
    ut = jnp.asarray(u_np.T, dtype=jnp.bfloat16)
    grid_spec = pltpu.PrefetchScalarGridSpec(
        num_scalar_prefetch=1,
        grid=(TOPK // MASKS_PER_STEP,),
        in_specs=(
            [pl.BlockSpec(
                (1, IN_HW, IN_HW),
                functools.partial(
                    lambda jj, i, qid_ref: (qid_ref[MASKS_PER_STEP * i + jj],
                                            0, 0), j))
             for j in range(MASKS_PER_STEP)] +
            [pl.BlockSpec((OUT_HW, IN_HW), lambda i, qid_ref: (0, 0)),
             pl.BlockSpec((IN_HW, OUT_HW), lambda i, qid_ref: (0, 0)),
             pl.BlockSpec((MASKS_PER_STEP, 1, 128),
                          lambda i, qid_ref: (i, 0, 0))]
        ),
        out_specs=[
            pl.BlockSpec((MASKS_PER_STEP, OUT_HW, OUT_HW),
                         lambda i, qid_ref: (i, 0, 0)),
            pl.BlockSpec((MASKS_PER_STEP, 1, 128),
                         lambda i, qid_ref: (i, 0, 0)),
        ],
    )
    pred, score = pl.pallas_call(
        _mask_kernel,
        grid_spec=grid_spec,
        out_shape=(
            jax.ShapeDtypeStruct((TOPK, OUT_HW, OUT_HW), jnp.bool_),
            jax.ShapeDtypeStruct((TOPK, 1, 128), jnp.float32),
        ),
    )(qid, *([mask_pred3d] * MASKS_PER_STEP), u, ut, tv)
    return pred, score[:, 0, 0]


def kernel(mask_cls, mask_pred):
    mask_cls2d = mask_cls.reshape(Q, NUM_TOTAL)
    vals3, labels, qid = _run_topk(mask_cls2d)
    pred_masks, final_scores = _run_masks(mask_pred[0], qid, vals3)
    return final_scores, labels, pred_masks
